# Initial kernel scaffold; baseline (speedup 1.0000x reference)
import jax
import jax.numpy as jnp
from jax.experimental import pallas as pl


def kernel(x, edge_index, W1, b1, W2, b2):
    raise NotImplementedError("write your pallas kernel here")


# speedup vs baseline: 13.6725x; 13.6725x over previous
import functools

import jax
import jax.numpy as jnp
from jax import lax
from jax.experimental import pallas as pl
from jax.experimental.pallas import tpu as pltpu
from jax.experimental.pallas import tpu_sc as plsc

N = 10000
E = 320000
DIN = 128
DHID = 128
DOUT = 16

NC, NS = 2, 16
NW = NC * NS
CHUNK = 128
EPT = -(-E // NW // CHUNK) * CHUNK
NCHUNK = EPT // CHUNK
EPAD = EPT * NW
NPAD = 10240
RPT = NPAD // NS

_MESH = plsc.VectorSubcoreMesh(core_axis_name="c", subcore_axis_name="s")


@functools.partial(
    pl.kernel,
    out_type=jax.ShapeDtypeStruct((NC, NPAD), jnp.float32),
    mesh=_MESH,
    scratch_types=[
        pltpu.VMEM((CHUNK,), jnp.int32),
        pltpu.VMEM((CHUNK,), jnp.float32),
        pltpu.VMEM_SHARED((NPAD,), jnp.float32),
    ],
)
def _deg_kernel(dst_hbm, zeros_hbm, out_hbm, idx_v, ones_v, acc):
    c = lax.axis_index("c")
    s = lax.axis_index("s")
    for i in range(CHUNK // 16):
        ones_v[pl.ds(i * 16, 16)] = jnp.ones((16,), jnp.float32)
    rs = s * RPT
    pltpu.sync_copy(zeros_hbm.at[pl.ds(rs, RPT)], acc.at[pl.ds(rs, RPT)])
    plsc.subcore_barrier()
    base = (s * NC + c) * EPT

    def body(j, carry):
        off = base + j * CHUNK
        pltpu.sync_copy(dst_hbm.at[pl.ds(off, CHUNK)], idx_v)
        pltpu.sync_copy(ones_v, acc.at[idx_v], add=True)
        return carry

    lax.fori_loop(0, NCHUNK, body, None)
    plsc.subcore_barrier()
    pltpu.sync_copy(acc.at[pl.ds(rs, RPT)], out_hbm.at[c, pl.ds(rs, RPT)])


def _make_prop(D):
    @functools.partial(
        pl.kernel,
        out_type=jax.ShapeDtypeStruct((NC, NPAD, D), jnp.float32),
        mesh=_MESH,
        scratch_types=[
            pltpu.VMEM((CHUNK,), jnp.int32),
            pltpu.VMEM((CHUNK,), jnp.int32),
            pltpu.VMEM((CHUNK, D), jnp.float32),
            pltpu.SemaphoreType.DMA,
            pltpu.VMEM_SHARED((NPAD, D), jnp.float32),
        ],
        compiler_params=pltpu.CompilerParams(use_tc_tiling_on_sc=False),
    )
    def prop(tab_hbm, src_hbm, dst_hbm, zeros_hbm, out_hbm,
             sidx, didx, rows, sem, acc):
        c = lax.axis_index("c")
        s = lax.axis_index("s")
        rs = s * RPT

        @pl.when(c == 0)
        def _():
            pltpu.sync_copy(tab_hbm.at[pl.ds(rs, RPT)], acc.at[pl.ds(rs, RPT)])

        @pl.when(c == 1)
        def _():
            pltpu.sync_copy(zeros_hbm.at[pl.ds(rs, RPT)],
                            acc.at[pl.ds(rs, RPT)])

        plsc.subcore_barrier()
        base = (s * NC + c) * EPT

        def body(j, carry):
            off = base + j * CHUNK
            pltpu.sync_copy(src_hbm.at[pl.ds(off, CHUNK)], sidx)
            pltpu.async_copy(tab_hbm.at[sidx], rows, sem).wait()
            pltpu.sync_copy(dst_hbm.at[pl.ds(off, CHUNK)], didx)
            pltpu.sync_copy(rows, acc.at[didx], add=True)
            return carry

        lax.fori_loop(0, NCHUNK, body, None)
        plsc.subcore_barrier()
        pltpu.sync_copy(acc.at[pl.ds(rs, RPT)], out_hbm.at[c, pl.ds(rs, RPT)])

    return prop


_prop128 = _make_prop(DHID)
_prop16 = _make_prop(DOUT)


BLK = 400


def _scale_mm_body(x_ref, w_ref, d0, d1, hhat_ref, dinv_ref):
    deg = d0[0] + d1[0] + 1.0
    dinv = lax.rsqrt(deg)
    h = jnp.dot(x_ref[...], w_ref[...], preferred_element_type=jnp.float32)
    hhat_ref[...] = h * dinv
    dinv_ref[...] = dinv


def _scale_mm(x, w, deg3):
    return pl.pallas_call(
        _scale_mm_body,
        grid=(N // BLK,),
        in_specs=[
            pl.BlockSpec((BLK, DIN), lambda i: (i, 0)),
            pl.BlockSpec((DIN, DHID), lambda i: (0, 0)),
            pl.BlockSpec((1, BLK, 1), lambda i: (0, i, 0)),
            pl.BlockSpec((1, BLK, 1), lambda i: (1, i, 0)),
        ],
        out_specs=[
            pl.BlockSpec((BLK, DHID), lambda i: (i, 0)),
            pl.BlockSpec((BLK, 1), lambda i: (i, 0)),
        ],
        out_shape=[
            jax.ShapeDtypeStruct((NPAD, DHID), jnp.float32),
            jax.ShapeDtypeStruct((N, 1), jnp.float32),
        ],
    )(x, w, deg3, deg3)


def _mid_body(a0, a1, dinv_ref, w2_ref, b1_ref, hhat2_ref):
    s = a0[0] + a1[0]
    out1 = s * dinv_ref[...] + b1_ref[...]
    h2 = jnp.dot(out1, w2_ref[...], preferred_element_type=jnp.float32)
    hhat2_ref[...] = h2 * dinv_ref[...]


def _mid(acc1, dinv, w2, b1row):
    return pl.pallas_call(
        _mid_body,
        grid=(N // BLK,),
        in_specs=[
            pl.BlockSpec((1, BLK, DHID), lambda i: (0, i, 0)),
            pl.BlockSpec((1, BLK, DHID), lambda i: (1, i, 0)),
            pl.BlockSpec((BLK, 1), lambda i: (i, 0)),
            pl.BlockSpec((DHID, DOUT), lambda i: (0, 0)),
            pl.BlockSpec((1, DHID), lambda i: (0, 0)),
        ],
        out_specs=pl.BlockSpec((BLK, DOUT), lambda i: (i, 0)),
        out_shape=jax.ShapeDtypeStruct((NPAD, DOUT), jnp.float32),
    )(acc1, acc1, dinv, w2, b1row)


def _final_body(a0, a1, dinv_ref, b2_ref, out_ref):
    out_ref[...] = (a0[0] + a1[0]) * dinv_ref[...] + b2_ref[...]


def _final(acc2, dinv, b2row):
    return pl.pallas_call(
        _final_body,
        grid=(N // BLK,),
        in_specs=[
            pl.BlockSpec((1, BLK, DOUT), lambda i: (0, i, 0)),
            pl.BlockSpec((1, BLK, DOUT), lambda i: (1, i, 0)),
            pl.BlockSpec((BLK, 1), lambda i: (i, 0)),
            pl.BlockSpec((1, DOUT), lambda i: (0, 0)),
        ],
        out_specs=pl.BlockSpec((BLK, DOUT), lambda i: (i, 0)),
        out_shape=jax.ShapeDtypeStruct((N, DOUT), jnp.float32),
    )(acc2, acc2, dinv, b2row)


def kernel(x, edge_index, W1, b1, W2, b2):
    src = edge_index[0].astype(jnp.int32)
    dst = edge_index[1].astype(jnp.int32)
    pad = EPAD - E
    src_p = jnp.concatenate([src, jnp.zeros((pad,), jnp.int32)])
    dst_p = jnp.concatenate([dst, jnp.full((pad,), N, jnp.int32)])
    zeros_n = jnp.zeros((NPAD,), jnp.float32)
    zeros_h = jnp.zeros((NPAD, DHID), jnp.float32)
    zeros_o = jnp.zeros((NPAD, DOUT), jnp.float32)

    deg_parts = _deg_kernel(dst_p, zeros_n)
    deg3 = deg_parts.reshape(NC, NPAD, 1)
    hhat1, dinv = _scale_mm(x, W1, deg3)
    acc1 = _prop128(hhat1, src_p, dst_p, zeros_h)
    hhat2 = _mid(acc1, dinv, W2, b1.reshape(1, DHID))
    acc2 = _prop16(hhat2, src_p, dst_p, zeros_o)
    return _final(acc2, dinv, b2.reshape(1, DOUT))
